# baseline (speedup 1.0000x reference)
import functools
import jax
import jax.numpy as jnp
from jax import lax
from jax.experimental import pallas as pl
from jax.experimental.pallas import tpu as pltpu
from jax.experimental.pallas import tpu_sc as plsc

_B, _S, _H = 1, 2048, 2048
_FFN = 8192
_R = 16
_E = 8
_TOPK = 2
_T = _B * _S

_NSLOT = _T * _TOPK
_BK = 256
_NB = _NSLOT // _BK + _E
_NSLOTP = _NB * _BK
_FILL = _NSLOTP - _NSLOT
_TE = _T * _E
_FB = 4096

_NTILE = 32
_JCHUNK = _NSLOTP // _NTILE


def _dotT(a, b):
    return jax.lax.dot_general(a, b, (((1,), (1,)), ((), ())),
                               preferred_element_type=jnp.float32)


def _dot(a, b):
    return jax.lax.dot_general(a, b, (((1,), (0,)), ((), ())),
                               preferred_element_type=jnp.float32)


def _a_body(hs_ref, gate_ref, w1a_ref, w3a_ref,
            rw_ref, u1_ref, u3w_ref, pos_ref, rowid_ref, tok_ref, eob_ref):
    hs = hs_ref[...]

    logits = _dotT(hs, gate_ref[...])
    m = jnp.max(logits, axis=1, keepdims=True)
    p = jnp.exp(logits - m)
    p = p / jnp.sum(p, axis=1, keepdims=True)
    lane = jax.lax.broadcasted_iota(jnp.int32, (_T, _E), 1)
    i1 = jnp.argmax(p, axis=1).astype(jnp.int32).reshape(_T, 1)
    w1 = jnp.max(p, axis=1, keepdims=True)
    p2 = jnp.where(lane == i1, -1.0, p)
    i2 = jnp.argmax(p2, axis=1).astype(jnp.int32).reshape(_T, 1)
    w2 = jnp.max(p2, axis=1, keepdims=True)
    s = w1 + w2
    w1n = w1 / s
    w2n = w2 / s
    combine = (jnp.where(lane == i1, w1n, 0.0)
               + jnp.where(lane == i2, w2n, 0.0))
    rw_ref[...] = jnp.concatenate([w1n, w2n], axis=1)

    u1_ref[...] = _dotT(hs, w1a_ref[...])
    u3 = _dotT(hs, w3a_ref[...])
    lane128 = jax.lax.broadcasted_iota(jnp.int32, (_E, _E * _R), 1)
    row8 = jax.lax.broadcasted_iota(jnp.int32, (_E, _E * _R), 0)
    spread = (lane128 // _R == row8).astype(jnp.float32)
    cexp = _dot(combine, spread)
    u3w_ref[...] = u3 * cexp

    eids = jnp.concatenate([i1, i2], axis=0)
    tvec = jnp.concatenate(
        [jax.lax.broadcasted_iota(jnp.int32, (_T, 1), 0)] * 2, axis=0)
    rowid = tvec * _E + eids
    lane_e = jax.lax.broadcasted_iota(jnp.int32, (_NSLOT, _E), 1)
    onehot = (lane_e == eids).astype(jnp.float32)
    csum = onehot
    sh = 1
    while sh < _NSLOT:
        shifted = jnp.concatenate(
            [jnp.zeros((sh, _E), jnp.float32), csum[:-sh, :]], axis=0)
        csum = csum + shifted
        sh *= 2
    rank = jnp.sum(onehot * csum, axis=1, keepdims=True) - 1.0
    counts = csum[_NSLOT - 1:_NSLOT, :]
    nb = jnp.floor((counts + (_BK - 1)) * (1.0 / _BK))
    r8 = jax.lax.broadcasted_iota(jnp.int32, (_E, _E), 0)
    c8 = jax.lax.broadcasted_iota(jnp.int32, (_E, _E), 1)
    strict_lower = (r8 < c8).astype(jnp.float32)
    lower_incl = (r8 <= c8).astype(jnp.float32)
    bstart = _dot(nb, strict_lower)
    bend = bstart + nb
    slot_start = jnp.sum(onehot * bstart, axis=1, keepdims=True) * _BK
    pos_real = slot_start + rank

    padcount = nb * _BK - counts
    padcum = _dot(padcount, lower_incl)
    padtotal = padcum[0:1, _E - 1:_E]
    bend7 = bend[0:1, _E - 1:_E]
    qcol = jax.lax.broadcasted_iota(jnp.int32, (_FILL, 1), 0).astype(
        jnp.float32)
    eq = jnp.sum((qcol >= padcum).astype(jnp.float32), axis=1, keepdims=True)
    lane8f = jax.lax.broadcasted_iota(jnp.int32, (_FILL, _E), 1)
    ohq = (lane8f == eq.astype(jnp.int32)).astype(jnp.float32)
    base_q = jnp.sum(ohq * (bstart * _BK + counts), axis=1, keepdims=True)
    pexcl_q = jnp.sum(ohq * (padcum - padcount), axis=1, keepdims=True)
    pos_fill = jnp.where(eq < _E, base_q + (qcol - pexcl_q),
                         bend7 * _BK + (qcol - padtotal))

    pos_ref[...] = jnp.concatenate([pos_real, pos_fill],
                                   axis=0).astype(jnp.int32)
    rowid_ref[...] = jnp.concatenate(
        [rowid, jnp.zeros((_FILL, 1), jnp.int32)], axis=0)
    tok_ref[...] = jnp.concatenate(
        [tvec, jnp.full((_FILL, 1), _T, jnp.int32)], axis=0)

    gidx = jax.lax.broadcasted_iota(jnp.int32, (1, _NB), 1).astype(jnp.float32)
    eob = jnp.zeros((1, _NB), jnp.float32)
    for e in range(_E):
        eob = eob + (gidx >= bend[0:1, e:e + 1]).astype(jnp.float32)
    eob_ref[...] = eob.astype(jnp.int32)


def _run_a(hs, gate_w, w1a, w3a):
    full = lambda shape: pl.BlockSpec(shape, lambda: (0,) * len(shape))
    return pl.pallas_call(
        _a_body,
        in_specs=[full((_T, _H)), full((_E, _H)),
                  full((_E * _R, _H)), full((_E * _R, _H))],
        out_specs=[full((_T, _TOPK)), full((_T, _E * _R)), full((_T, _E * _R)),
                   full((_NSLOTP, 1)), full((_NSLOTP, 1)), full((_NSLOTP, 1)),
                   full((1, _NB))],
        out_shape=[
            jax.ShapeDtypeStruct((_T, _TOPK), jnp.float32),
            jax.ShapeDtypeStruct((_T, _E * _R), jnp.float32),
            jax.ShapeDtypeStruct((_T, _E * _R), jnp.float32),
            jax.ShapeDtypeStruct((_NSLOTP, 1), jnp.int32),
            jax.ShapeDtypeStruct((_NSLOTP, 1), jnp.int32),
            jax.ShapeDtypeStruct((_NSLOTP, 1), jnp.int32),
            jax.ShapeDtypeStruct((1, _NB), jnp.int32),
        ],
    )(hs, gate_w, w1a, w3a)


def _sck1_body(pos_hbm, rowid_hbm, tok_hbm, u1flat_hbm, u3wflat_hbm,
               tokbrd_hbm, u1g_hbm, u3wg_hbm, tokg_hbm,
               pos_vm, idx_vm, tok_vm, rows1_vm, rows3_vm, rowst_vm, sem):
    wid = lax.axis_index("s") * 2 + lax.axis_index("c")
    base = wid * _JCHUNK

    pltpu.sync_copy(pos_hbm.at[pl.ds(base, _JCHUNK)], pos_vm)
    pltpu.sync_copy(rowid_hbm.at[pl.ds(base, _JCHUNK)], idx_vm)
    pltpu.sync_copy(tok_hbm.at[pl.ds(base, _JCHUNK)], tok_vm)
    pltpu.async_copy(u1flat_hbm.at[idx_vm], rows1_vm, sem).wait()
    pltpu.sync_copy(rows1_vm, u1g_hbm.at[pos_vm])
    pltpu.async_copy(u3wflat_hbm.at[idx_vm], rows3_vm, sem).wait()
    pltpu.sync_copy(rows3_vm, u3wg_hbm.at[pos_vm])
    pltpu.async_copy(tokbrd_hbm.at[tok_vm], rowst_vm, sem).wait()
    pltpu.sync_copy(rowst_vm, tokg_hbm.at[pos_vm])


def _run_sck1(pos, rowid, tok, u1flat, u3wflat, tokbrd):
    mesh = plsc.VectorSubcoreMesh(core_axis_name="c", subcore_axis_name="s")
    return pl.kernel(
        _sck1_body,
        out_type=[
            jax.ShapeDtypeStruct((_NSLOTP, _R), jnp.float32),
            jax.ShapeDtypeStruct((_NSLOTP, _R), jnp.float32),
            jax.ShapeDtypeStruct((_NSLOTP, _R), jnp.int32),
        ],
        mesh=mesh,
        compiler_params=pltpu.CompilerParams(use_tc_tiling_on_sc=False),
        scratch_types=[
            pltpu.VMEM((_JCHUNK,), jnp.int32),
            pltpu.VMEM((_JCHUNK,), jnp.int32),
            pltpu.VMEM((_JCHUNK,), jnp.int32),
            pltpu.VMEM((_JCHUNK, _R), jnp.float32),
            pltpu.VMEM((_JCHUNK, _R), jnp.float32),
            pltpu.VMEM((_JCHUNK, _R), jnp.int32),
            pltpu.SemaphoreType.DMA,
        ],
    )(pos, rowid, tok, u1flat, u3wflat, tokbrd)


def _c_body(eob_ref, u1_ref, u3_ref, tok_ref, w1b_ref, w3b_ref, w2a_ref,
            w2b_ref, out_ref, zacc_ref):
    g = pl.program_id(0)
    e = eob_ref[g]

    @pl.when(g == 0)
    def _():
        zacc_ref[...] = jnp.zeros((_T, _E * _R), jnp.float32)

    @pl.when(e < _E)
    def _():
        u1 = u1_ref[...]
        u3 = u3_ref[...]
        acc = jnp.zeros((_BK, _R), jnp.float32)
        for f in range(_FFN // _FB):
            w1b = w1b_ref[e, :, f * _FB:(f + 1) * _FB]
            w3b = w3b_ref[e, :, f * _FB:(f + 1) * _FB]
            w2a = w2a_ref[e, :, f * _FB:(f + 1) * _FB]
            a1 = _dot(u1, w1b)
            a3 = _dot(u3, w3b)
            inter = jnp.maximum(a1, 0.01 * a1) * a3
            acc = acc + _dotT(inter, w2a)

        tok = tok_ref[...][:, 0:1]
        p2 = (jax.lax.broadcasted_iota(jnp.int32, (_BK, _T), 1)
              == tok).astype(jnp.bfloat16)
        row128 = jax.lax.broadcasted_iota(jnp.int32, (_E * _R, _R), 0)
        lane16 = jax.lax.broadcasted_iota(jnp.int32, (_E * _R, _R), 1)
        sel = (row128 == e * _R + lane16).astype(jnp.bfloat16)
        accs = _dotT(acc.astype(jnp.bfloat16), sel)
        contrib = jax.lax.dot_general(
            p2, accs.astype(jnp.bfloat16), (((0,), (0,)), ((), ())),
            preferred_element_type=jnp.float32)
        zacc_ref[...] += contrib

    @pl.when(g == _NB - 1)
    def _():
        out_ref[...] = _dot(zacc_ref[...].astype(jnp.bfloat16), w2b_ref[...])


def _run_c(eob, u1g, u3wg, tokg, w1b, w3b, w2a, w2b_bf):
    grid_spec = pltpu.PrefetchScalarGridSpec(
        num_scalar_prefetch=1,
        grid=(_NB,),
        in_specs=[
            pl.BlockSpec((_BK, _R), lambda g, eob_s: (g, 0)),
            pl.BlockSpec((_BK, _R), lambda g, eob_s: (g, 0)),
            pl.BlockSpec((_BK, _R), lambda g, eob_s: (g, 0)),
            pl.BlockSpec((_E, _R, _FFN), lambda g, eob_s: (0, 0, 0)),
            pl.BlockSpec((_E, _R, _FFN), lambda g, eob_s: (0, 0, 0)),
            pl.BlockSpec((_E, _R, _FFN), lambda g, eob_s: (0, 0, 0)),
            pl.BlockSpec((_E * _R, _H), lambda g, eob_s: (0, 0)),
        ],
        out_specs=pl.BlockSpec((_T, _H), lambda g, eob_s: (0, 0)),
        scratch_shapes=[pltpu.VMEM((_T, _E * _R), jnp.float32)],
    )
    return pl.pallas_call(
        _c_body,
        grid_spec=grid_spec,
        out_shape=jax.ShapeDtypeStruct((_T, _H), jnp.float32),
        compiler_params=pltpu.CompilerParams(
            dimension_semantics=("arbitrary",),
        ),
    )(eob, u1g, u3wg, tokg, w1b, w3b, w2a, w2b_bf)


@jax.jit
def kernel(hidden_states, gate_w, w1_A, w1_B, w2_A, w2_B, w3_A, w3_B):
    hs = hidden_states.reshape(_T, _H)
    w1a = w1_A.reshape(_E * _R, _H)
    w3a = w3_A.reshape(_E * _R, _H)
    w2b = w2_B.transpose(0, 2, 1).reshape(_E * _R, _H)
    w1b = w1_B.transpose(0, 2, 1)
    w3b = w3_B.transpose(0, 2, 1)

    rw, u1, u3w, pos2, rowid2, tok2, eob2 = _run_a(hs, gate_w, w1a, w3a)

    pos = pos2.reshape(_NSLOTP)
    rowid = rowid2.reshape(_NSLOTP)
    tok = tok2.reshape(_NSLOTP)
    eob = eob2.reshape(_NB)
    u1flat = u1.reshape(_TE, _R)
    u3wflat = u3w.reshape(_TE, _R)
    tokbrd = jnp.broadcast_to(
        jax.lax.iota(jnp.int32, _T + 8)[:, None], (_T + 8, _R))

    u1g, u3wg, tokg = _run_sck1(pos, rowid, tok, u1flat, u3wflat, tokbrd)
    out = _run_c(eob, u1g, u3wg, tokg, w1b, w3b, w2_A,
                 w2b.astype(jnp.bfloat16))
    return out.reshape(_B, _S, _H), rw


# speedup vs baseline: 1.0563x; 1.0563x over previous
import functools
import jax
import jax.numpy as jnp
from jax import lax
from jax.experimental import pallas as pl
from jax.experimental.pallas import tpu as pltpu
from jax.experimental.pallas import tpu_sc as plsc

_B, _S, _H = 1, 2048, 2048
_FFN = 8192
_R = 16
_E = 8
_TOPK = 2
_T = _B * _S

_NSLOT = _T * _TOPK
_BK = 256
_NB = _NSLOT // _BK + _E
_NSLOTP = _NB * _BK
_FILL = _NSLOTP - _NSLOT
_TE = _T * _E
_FB = 8192

_NTILE = 32
_JCHUNK = _NSLOTP // _NTILE


def _dotT(a, b):
    return jax.lax.dot_general(a, b, (((1,), (1,)), ((), ())),
                               preferred_element_type=jnp.float32)


def _dot(a, b):
    return jax.lax.dot_general(a, b, (((1,), (0,)), ((), ())),
                               preferred_element_type=jnp.float32)


def _a_body(hs_ref, gate_ref, w1a_ref, w3a_ref,
            rw_ref, u1_ref, u3w_ref, pos_ref, rowid_ref, tok_ref, eob_ref):
    hs = hs_ref[...]

    logits = _dotT(hs, gate_ref[...])
    m = jnp.max(logits, axis=1, keepdims=True)
    p = jnp.exp(logits - m)
    p = p / jnp.sum(p, axis=1, keepdims=True)
    lane = jax.lax.broadcasted_iota(jnp.int32, (_T, _E), 1)
    i1 = jnp.argmax(p, axis=1).astype(jnp.int32).reshape(_T, 1)
    w1 = jnp.max(p, axis=1, keepdims=True)
    p2 = jnp.where(lane == i1, -1.0, p)
    i2 = jnp.argmax(p2, axis=1).astype(jnp.int32).reshape(_T, 1)
    w2 = jnp.max(p2, axis=1, keepdims=True)
    s = w1 + w2
    w1n = w1 / s
    w2n = w2 / s
    combine = (jnp.where(lane == i1, w1n, 0.0)
               + jnp.where(lane == i2, w2n, 0.0))
    rw_ref[...] = jnp.concatenate([w1n, w2n], axis=1)

    u1_ref[...] = _dotT(hs, w1a_ref[...])
    u3 = _dotT(hs, w3a_ref[...])
    lane128 = jax.lax.broadcasted_iota(jnp.int32, (_E, _E * _R), 1)
    row8 = jax.lax.broadcasted_iota(jnp.int32, (_E, _E * _R), 0)
    spread = (lane128 // _R == row8).astype(jnp.float32)
    cexp = _dot(combine, spread)
    u3w_ref[...] = u3 * cexp

    eids = jnp.concatenate([i1, i2], axis=0)
    tvec = jnp.concatenate(
        [jax.lax.broadcasted_iota(jnp.int32, (_T, 1), 0)] * 2, axis=0)
    rowid = tvec * _E + eids
    lane_e = jax.lax.broadcasted_iota(jnp.int32, (_NSLOT, _E), 1)
    onehot = (lane_e == eids).astype(jnp.float32)
    csum = onehot
    sh = 1
    while sh < _NSLOT:
        shifted = jnp.concatenate(
            [jnp.zeros((sh, _E), jnp.float32), csum[:-sh, :]], axis=0)
        csum = csum + shifted
        sh *= 2
    rank = jnp.sum(onehot * csum, axis=1, keepdims=True) - 1.0
    counts = csum[_NSLOT - 1:_NSLOT, :]
    nb = jnp.floor((counts + (_BK - 1)) * (1.0 / _BK))
    r8 = jax.lax.broadcasted_iota(jnp.int32, (_E, _E), 0)
    c8 = jax.lax.broadcasted_iota(jnp.int32, (_E, _E), 1)
    strict_lower = (r8 < c8).astype(jnp.float32)
    lower_incl = (r8 <= c8).astype(jnp.float32)
    bstart = _dot(nb, strict_lower)
    bend = bstart + nb
    slot_start = jnp.sum(onehot * bstart, axis=1, keepdims=True) * _BK
    pos_real = slot_start + rank

    padcount = nb * _BK - counts
    padcum = _dot(padcount, lower_incl)
    padtotal = padcum[0:1, _E - 1:_E]
    bend7 = bend[0:1, _E - 1:_E]
    qcol = jax.lax.broadcasted_iota(jnp.int32, (_FILL, 1), 0).astype(
        jnp.float32)
    eq = jnp.sum((qcol >= padcum).astype(jnp.float32), axis=1, keepdims=True)
    lane8f = jax.lax.broadcasted_iota(jnp.int32, (_FILL, _E), 1)
    ohq = (lane8f == eq.astype(jnp.int32)).astype(jnp.float32)
    base_q = jnp.sum(ohq * (bstart * _BK + counts), axis=1, keepdims=True)
    pexcl_q = jnp.sum(ohq * (padcum - padcount), axis=1, keepdims=True)
    pos_fill = jnp.where(eq < _E, base_q + (qcol - pexcl_q),
                         bend7 * _BK + (qcol - padtotal))

    pos_ref[...] = jnp.concatenate([pos_real, pos_fill],
                                   axis=0).astype(jnp.int32)
    rowid_ref[...] = jnp.concatenate(
        [rowid, jnp.zeros((_FILL, 1), jnp.int32)], axis=0)
    tok_ref[...] = jnp.concatenate(
        [tvec, jnp.full((_FILL, 1), _T, jnp.int32)], axis=0)

    gidx = jax.lax.broadcasted_iota(jnp.int32, (1, _NB), 1).astype(jnp.float32)
    eob = jnp.zeros((1, _NB), jnp.float32)
    for e in range(_E):
        eob = eob + (gidx >= bend[0:1, e:e + 1]).astype(jnp.float32)
    eob_ref[...] = eob.astype(jnp.int32)


def _run_a(hs, gate_w, w1a, w3a):
    full = lambda shape: pl.BlockSpec(shape, lambda: (0,) * len(shape))
    return pl.pallas_call(
        _a_body,
        in_specs=[full((_T, _H)), full((_E, _H)),
                  full((_E * _R, _H)), full((_E * _R, _H))],
        out_specs=[full((_T, _TOPK)), full((_T, _E * _R)), full((_T, _E * _R)),
                   full((_NSLOTP, 1)), full((_NSLOTP, 1)), full((_NSLOTP, 1)),
                   full((1, _NB))],
        out_shape=[
            jax.ShapeDtypeStruct((_T, _TOPK), jnp.float32),
            jax.ShapeDtypeStruct((_T, _E * _R), jnp.float32),
            jax.ShapeDtypeStruct((_T, _E * _R), jnp.float32),
            jax.ShapeDtypeStruct((_NSLOTP, 1), jnp.int32),
            jax.ShapeDtypeStruct((_NSLOTP, 1), jnp.int32),
            jax.ShapeDtypeStruct((_NSLOTP, 1), jnp.int32),
            jax.ShapeDtypeStruct((1, _NB), jnp.int32),
        ],
    )(hs, gate_w, w1a, w3a)


def _sck1_body(pos_hbm, rowid_hbm, tok_hbm, u1flat_hbm, u3wflat_hbm,
               tokbrd_hbm, u1g_hbm, u3wg_hbm, tokg_hbm,
               pos_vm, idx_vm, tok_vm, rows1_vm, rows3_vm, rowst_vm, sem):
    wid = lax.axis_index("s") * 2 + lax.axis_index("c")
    base = wid * _JCHUNK

    pltpu.sync_copy(pos_hbm.at[pl.ds(base, _JCHUNK)], pos_vm)
    pltpu.sync_copy(rowid_hbm.at[pl.ds(base, _JCHUNK)], idx_vm)
    pltpu.sync_copy(tok_hbm.at[pl.ds(base, _JCHUNK)], tok_vm)
    pltpu.async_copy(u1flat_hbm.at[idx_vm], rows1_vm, sem).wait()
    pltpu.sync_copy(rows1_vm, u1g_hbm.at[pos_vm])
    pltpu.async_copy(u3wflat_hbm.at[idx_vm], rows3_vm, sem).wait()
    pltpu.sync_copy(rows3_vm, u3wg_hbm.at[pos_vm])
    pltpu.async_copy(tokbrd_hbm.at[tok_vm], rowst_vm, sem).wait()
    pltpu.sync_copy(rowst_vm, tokg_hbm.at[pos_vm])


def _run_sck1(pos, rowid, tok, u1flat, u3wflat, tokbrd):
    mesh = plsc.VectorSubcoreMesh(core_axis_name="c", subcore_axis_name="s")
    return pl.kernel(
        _sck1_body,
        out_type=[
            jax.ShapeDtypeStruct((_NSLOTP, _R), jnp.float32),
            jax.ShapeDtypeStruct((_NSLOTP, _R), jnp.float32),
            jax.ShapeDtypeStruct((_NSLOTP, _R), jnp.int32),
        ],
        mesh=mesh,
        compiler_params=pltpu.CompilerParams(use_tc_tiling_on_sc=False),
        scratch_types=[
            pltpu.VMEM((_JCHUNK,), jnp.int32),
            pltpu.VMEM((_JCHUNK,), jnp.int32),
            pltpu.VMEM((_JCHUNK,), jnp.int32),
            pltpu.VMEM((_JCHUNK, _R), jnp.float32),
            pltpu.VMEM((_JCHUNK, _R), jnp.float32),
            pltpu.VMEM((_JCHUNK, _R), jnp.int32),
            pltpu.SemaphoreType.DMA,
        ],
    )(pos, rowid, tok, u1flat, u3wflat, tokbrd)


def _c_body(eob_ref, u1_ref, u3_ref, tok_ref, w1b_ref, w3b_ref, w2a_ref,
            w2b_ref, out_ref, zacc_ref):
    g = pl.program_id(0)
    e = eob_ref[g]

    @pl.when(g == 0)
    def _():
        zacc_ref[...] = jnp.zeros((_T, _E * _R), jnp.float32)

    @pl.when(e < _E)
    def _():
        u1 = u1_ref[...]
        u3 = u3_ref[...]
        acc = jnp.zeros((_BK, _R), jnp.float32)
        for f in range(_FFN // _FB):
            w1b = w1b_ref[e, :, f * _FB:(f + 1) * _FB]
            w3b = w3b_ref[e, :, f * _FB:(f + 1) * _FB]
            w2a = w2a_ref[e, :, f * _FB:(f + 1) * _FB]
            a1 = _dot(u1, w1b)
            a3 = _dot(u3, w3b)
            inter = jnp.maximum(a1, 0.01 * a1) * a3
            acc = acc + _dotT(inter, w2a)

        tok = tok_ref[...][:, 0:1]
        p2 = (jax.lax.broadcasted_iota(jnp.int32, (_BK, _T), 1)
              == tok).astype(jnp.bfloat16)
        row128 = jax.lax.broadcasted_iota(jnp.int32, (_E * _R, _R), 0)
        lane16 = jax.lax.broadcasted_iota(jnp.int32, (_E * _R, _R), 1)
        sel = (row128 == e * _R + lane16).astype(jnp.bfloat16)
        accs = _dotT(acc.astype(jnp.bfloat16), sel)
        contrib = jax.lax.dot_general(
            p2, accs.astype(jnp.bfloat16), (((0,), (0,)), ((), ())),
            preferred_element_type=jnp.float32)
        zacc_ref[...] += contrib

    @pl.when(g == _NB - 1)
    def _():
        out_ref[...] = _dot(zacc_ref[...].astype(jnp.bfloat16), w2b_ref[...])


def _run_c(eob, u1g, u3wg, tokg, w1b, w3b, w2a, w2b_bf):
    grid_spec = pltpu.PrefetchScalarGridSpec(
        num_scalar_prefetch=1,
        grid=(_NB,),
        in_specs=[
            pl.BlockSpec((_BK, _R), lambda g, eob_s: (g, 0)),
            pl.BlockSpec((_BK, _R), lambda g, eob_s: (g, 0)),
            pl.BlockSpec((_BK, _R), lambda g, eob_s: (g, 0)),
            pl.BlockSpec((_E, _R, _FFN), lambda g, eob_s: (0, 0, 0)),
            pl.BlockSpec((_E, _R, _FFN), lambda g, eob_s: (0, 0, 0)),
            pl.BlockSpec((_E, _R, _FFN), lambda g, eob_s: (0, 0, 0)),
            pl.BlockSpec((_E * _R, _H), lambda g, eob_s: (0, 0)),
        ],
        out_specs=pl.BlockSpec((_T, _H), lambda g, eob_s: (0, 0)),
        scratch_shapes=[pltpu.VMEM((_T, _E * _R), jnp.float32)],
    )
    return pl.pallas_call(
        _c_body,
        grid_spec=grid_spec,
        out_shape=jax.ShapeDtypeStruct((_T, _H), jnp.float32),
        compiler_params=pltpu.CompilerParams(
            dimension_semantics=("arbitrary",),
        ),
    )(eob, u1g, u3wg, tokg, w1b, w3b, w2a, w2b_bf)


@jax.jit
def kernel(hidden_states, gate_w, w1_A, w1_B, w2_A, w2_B, w3_A, w3_B):
    hs = hidden_states.reshape(_T, _H)
    w1a = w1_A.reshape(_E * _R, _H)
    w3a = w3_A.reshape(_E * _R, _H)
    w2b = w2_B.transpose(0, 2, 1).reshape(_E * _R, _H)
    w1b = w1_B.transpose(0, 2, 1)
    w3b = w3_B.transpose(0, 2, 1)

    rw, u1, u3w, pos2, rowid2, tok2, eob2 = _run_a(hs, gate_w, w1a, w3a)

    pos = pos2.reshape(_NSLOTP)
    rowid = rowid2.reshape(_NSLOTP)
    tok = tok2.reshape(_NSLOTP)
    eob = eob2.reshape(_NB)
    u1flat = u1.reshape(_TE, _R)
    u3wflat = u3w.reshape(_TE, _R)
    tokbrd = jnp.broadcast_to(
        jax.lax.iota(jnp.int32, _T + 8)[:, None], (_T + 8, _R))

    u1g, u3wg, tokg = _run_sck1(pos, rowid, tok, u1flat, u3wflat, tokbrd)
    out = _run_c(eob, u1g, u3wg, tokg, w1b, w3b, w2_A,
                 w2b.astype(jnp.bfloat16))
    return out.reshape(_B, _S, _H), rw
